# baseline (speedup 1.0000x reference)
import functools

import jax
import jax.numpy as jnp
from jax import lax
from jax.experimental import pallas as pl
from jax.experimental.pallas import tpu as pltpu
from jax.experimental.pallas import tpu_sc as plsc

_D = 1024
_E = 16
_FF = 2048
_K = 2
_NTOK = 4096
_NSLOT = _K * _NTOK
_T = 64
_NPAD = _NSLOT + _E * _T
_NT = _NPAD // _T
_TT = 256
_NTT = _NTOK // _TT
_LB_COEF = 0.01
_Z_COEF = 0.001

_NC = 2
_NS = 16


def _gate_body(x_ref, g_ref, b_ref, wg_ref,
               xn_ref, e0_ref, e1_ref, w0_ref, w1_ref, acc_ref):
    t = pl.program_id(0)
    xt = x_ref[...]
    mu = jnp.mean(xt, axis=1, keepdims=True)
    var = jnp.mean((xt - mu) ** 2, axis=1, keepdims=True)
    xn = g_ref[...] * (xt - mu) / jnp.sqrt(var + 1e-5) + b_ref[...]
    xn_ref[...] = xn
    logits = lax.dot_general(xn, wg_ref[...], (((1,), (1,)), ((), ())),
                             preferred_element_type=jnp.float32)
    logits = jnp.clip(logits, -10.0, 10.0)
    lm = jnp.max(logits, axis=1, keepdims=True)
    ex = jnp.exp(logits - lm)
    se = jnp.sum(ex, axis=1, keepdims=True)
    probs = ex / se
    lse = jnp.log(se[:, 0]) + lm[:, 0]
    iota = lax.broadcasted_iota(jnp.int32, (_TT, _E), 1)
    m0 = jnp.max(probs, axis=1, keepdims=True)
    e0 = jnp.min(jnp.where(probs == m0, iota, _E), axis=1)
    pm = jnp.where(iota == e0[:, None], -1.0, probs)
    m1 = jnp.max(pm, axis=1, keepdims=True)
    e1 = jnp.min(jnp.where(pm == m1, iota, _E), axis=1)
    denom = m0[:, 0] + m1[:, 0] + 1e-8
    e0_ref[0, 0, :] = e0
    e1_ref[0, 0, :] = e1
    w0_ref[0, 0, :] = m0[:, 0] / denom
    w1_ref[0, 0, :] = m1[:, 0] / denom
    ps = jnp.sum(probs, axis=0)
    top1 = jnp.sum((iota == e0[:, None]).astype(jnp.float32), axis=0)
    zsq = jnp.sum(lse * lse)
    zpad = jnp.zeros((128 - _E,), jnp.float32)
    col = lax.broadcasted_iota(jnp.int32, (1, 128), 1)
    part = jnp.concatenate([
        jnp.concatenate([ps, zpad]).reshape(1, 128),
        jnp.concatenate([top1, zpad]).reshape(1, 128),
        jnp.where(col == 0, zsq, 0.0),
        jnp.zeros((5, 128), jnp.float32),
    ], axis=0)

    @pl.when(t == 0)
    def _():
        acc_ref[...] = part

    @pl.when(t != 0)
    def _():
        acc_ref[...] = acc_ref[...] + part


def _gate(x_flat, gamma, beta, wg):
    return pl.pallas_call(
        _gate_body,
        grid=(_NTT,),
        in_specs=[
            pl.BlockSpec((_TT, _D), lambda t: (t, 0)),
            pl.BlockSpec((1, _D), lambda t: (0, 0)),
            pl.BlockSpec((1, _D), lambda t: (0, 0)),
            pl.BlockSpec((_E, _D), lambda t: (0, 0)),
        ],
        out_specs=[
            pl.BlockSpec((_TT, _D), lambda t: (t, 0)),
            pl.BlockSpec((1, 1, _TT), lambda t: (t, 0, 0)),
            pl.BlockSpec((1, 1, _TT), lambda t: (t, 0, 0)),
            pl.BlockSpec((1, 1, _TT), lambda t: (t, 0, 0)),
            pl.BlockSpec((1, 1, _TT), lambda t: (t, 0, 0)),
            pl.BlockSpec((8, 128), lambda t: (0, 0)),
        ],
        out_shape=[
            jax.ShapeDtypeStruct((_NTOK, _D), jnp.float32),
            jax.ShapeDtypeStruct((_NTT, 1, _TT), jnp.int32),
            jax.ShapeDtypeStruct((_NTT, 1, _TT), jnp.int32),
            jax.ShapeDtypeStruct((_NTT, 1, _TT), jnp.float32),
            jax.ShapeDtypeStruct((_NTT, 1, _TT), jnp.float32),
            jax.ShapeDtypeStruct((8, 128), jnp.float32),
        ],
    )(x_flat, gamma.reshape(1, _D), beta.reshape(1, _D), wg)


_NW = _NC * _NS
_RS = _NSLOT // _NW
_DC = 32
_NDC = _RS // _DC


def _hist_body(ecat, hist_hbm, e_v, hist_v):
    wid = lax.axis_index("s") * _NC + lax.axis_index("c")
    pltpu.sync_copy(ecat.at[pl.ds(wid * _RS, _RS)], e_v)
    iota16 = lax.iota(jnp.int32, 16)
    lane15 = jnp.full((16,), 15, jnp.int32)

    def _splat_last(v):
        return jnp.take_along_axis(v, lane15, axis=0)

    def _h(j, hist):
        ev = e_v[pl.ds(j * 16, 16)]
        for b in range(_E):
            cs = plsc.cumsum(jnp.where(ev == b, 1, 0))
            hist = hist + jnp.where(iota16 == b, _splat_last(cs), 0)
        return hist

    hist = lax.fori_loop(0, _RS // 16, _h, jnp.zeros((16,), jnp.int32))
    hist_v[...] = hist
    pltpu.sync_copy(hist_v, hist_hbm.at[wid])


def _hist(ecat):
    mesh = plsc.VectorSubcoreMesh(
        core_axis_name="c", subcore_axis_name="s", num_cores=_NC,
        num_subcores=_NS)
    fn = pl.kernel(
        _hist_body,
        out_type=jax.ShapeDtypeStruct((_NW, 16), jnp.int32),
        mesh=mesh,
        scratch_types=[
            pltpu.VMEM((_RS,), jnp.int32),
            pltpu.VMEM((16,), jnp.int32),
        ],
        compiler_params=pltpu.CompilerParams(needs_layout_passes=False),
    )
    return fn(ecat)


def _dispatch_body(ecat, xn_hbm, hist_hbm, xs_hbm, inv_hbm,
                   e_v, pos_v, allh_v, posh_v, rows0, rows1,
                   l0, l1, s0, s1):
    wid = lax.axis_index("s") * _NC + lax.axis_index("c")
    base = wid * _RS
    pltpu.sync_copy(ecat.at[pl.ds(base, _RS)], e_v)
    pltpu.sync_copy(hist_hbm, allh_v)

    iota16 = lax.iota(jnp.int32, 16)
    lane15 = jnp.full((16,), 15, jnp.int32)

    def _splat_last(v):
        return jnp.take_along_axis(v, lane15, axis=0)

    def _s(r, carry):
        tot, pre = carry
        h = allh_v[r, :]
        m = jnp.where(r < wid, 1, 0)
        return tot + h, pre + h * m

    tot, pre = lax.fori_loop(
        0, _NW, _s,
        (jnp.zeros((16,), jnp.int32), jnp.zeros((16,), jnp.int32)))
    padded = ((tot + (_T - 1)) // _T) * _T
    offs_incl = plsc.cumsum(padded)
    start = offs_incl - padded + pre

    def _pz(j, _):
        pos_v[pl.ds(j * 16, 16)] = jnp.zeros((16,), jnp.int32)
        return 0
    lax.fori_loop(0, _RS // 16, _pz, 0)

    for b in range(_E):
        s_b = jnp.take_along_axis(start, jnp.full((16,), b, jnp.int32),
                                  axis=0)

        def _r(j, carry, b=b, s_b=s_b):
            sl = pl.ds(j * 16, 16)
            m = e_v[sl] == b
            cs = plsc.cumsum(jnp.where(m, 1, 0))
            pos_v[sl] = jnp.where(m, s_b + carry + cs - 1, pos_v[sl])
            return carry + _splat_last(cs)

        lax.fori_loop(0, _RS // 16, _r, jnp.zeros((16,), jnp.int32))

    for ch in range(_NDC):
        def _c(j, _, ch=ch):
            posh_v[ch, pl.ds(j * 16, 16)] = pos_v[pl.ds(ch * _DC + j * 16, 16)]
            return 0
        lax.fori_loop(0, _DC // 16, _c, 0)

    tokbase = base % _NTOK

    def _ld(ch, rows, sem):
        return pltpu.make_async_copy(
            xn_hbm.at[pl.ds(tokbase + ch * _DC, _DC)], rows, sem)

    def _sc(ch, rows, sem):
        return pltpu.make_async_copy(rows, xs_hbm.at[posh_v.at[ch]], sem)

    _ld(0, rows0, l0).start()
    _ld(1, rows1, l1).start()

    def _pair(q, _):
        c0 = 2 * q
        c1 = 2 * q + 1
        _ld(c0, rows0, l0).wait()
        _sc(c0, rows0, s0).start()
        _ld(c1, rows1, l1).wait()
        _sc(c1, rows1, s1).start()

        @pl.when(q < _NDC // 2 - 1)
        def _():
            _sc(c0, rows0, s0).wait()
            _ld(c0 + 2, rows0, l0).start()
            _sc(c1, rows1, s1).wait()
            _ld(c1 + 2, rows1, l1).start()
        return 0

    lax.fori_loop(0, _NDC // 2, _pair, 0)
    _sc(_NDC - 2, rows0, s0).wait()
    _sc(_NDC - 1, rows1, s1).wait()

    pltpu.sync_copy(pos_v, inv_hbm.at[pl.ds(base, _RS)])


def _dispatch(ecat, xn, hist):
    mesh = plsc.VectorSubcoreMesh(
        core_axis_name="c", subcore_axis_name="s", num_cores=_NC,
        num_subcores=_NS)
    fn = pl.kernel(
        _dispatch_body,
        out_type=(
            jax.ShapeDtypeStruct((_NPAD, _D), jnp.float32),
            jax.ShapeDtypeStruct((_NSLOT,), jnp.int32),
        ),
        mesh=mesh,
        scratch_types=[
            pltpu.VMEM((_RS,), jnp.int32),
            pltpu.VMEM((_RS,), jnp.int32),
            pltpu.VMEM((_NW, 16), jnp.int32),
            pltpu.VMEM((_NDC, _DC), jnp.int32),
            pltpu.VMEM((_DC, _D), jnp.float32),
            pltpu.VMEM((_DC, _D), jnp.float32),
            pltpu.SemaphoreType.DMA,
            pltpu.SemaphoreType.DMA,
            pltpu.SemaphoreType.DMA,
            pltpu.SemaphoreType.DMA,
        ],
        compiler_params=pltpu.CompilerParams(needs_layout_passes=False),
    )
    return fn(ecat, xn, hist)


def _ffn_body(te_ref, xs_ref, w1_ref, b1_ref, w2_ref, b2_ref, out_ref):
    del te_ref
    a = lax.dot_general(xs_ref[...], w1_ref[0], (((1,), (0,)), ((), ())),
                        preferred_element_type=jnp.float32)
    ap = a + b1_ref[0]
    h = 0.5 * ap * (1.0 + lax.erf(ap * 0.7071067811865476))
    o = lax.dot_general(h, w2_ref[0], (((1,), (0,)), ((), ())),
                        preferred_element_type=jnp.float32)
    out_ref[...] = o + b2_ref[0]


def _ffn(te, xs, w1, b1r, w2, b2r):
    grid_spec = pltpu.PrefetchScalarGridSpec(
        num_scalar_prefetch=1,
        grid=(_NT,),
        in_specs=[
            pl.BlockSpec((_T, _D), lambda t, te: (t, 0)),
            pl.BlockSpec((1, _D, _FF), lambda t, te: (te[t], 0, 0)),
            pl.BlockSpec((1, 1, _FF), lambda t, te: (te[t], 0, 0)),
            pl.BlockSpec((1, _FF, _D), lambda t, te: (te[t], 0, 0)),
            pl.BlockSpec((1, 1, _D), lambda t, te: (te[t], 0, 0)),
        ],
        out_specs=pl.BlockSpec((_T, _D), lambda t, te: (t, 0)),
    )
    return pl.pallas_call(
        _ffn_body,
        grid_spec=grid_spec,
        out_shape=jax.ShapeDtypeStruct((_NPAD, _D), jnp.float32),
    )(te, xs, w1, b1r, w2, b2r)


_CT = _NTOK // (_NC * _NS)
_CC = 16


def _combine_body(x_hbm, ob_hbm, inv_hbm, wcat_hbm, y_hbm,
                  i0_v, i1_v, w0_v, w1_v, x_v, r0_v, r1_v, y_v, sem):
    wid = lax.axis_index("s") * _NC + lax.axis_index("c")
    base = wid * _CT
    pltpu.sync_copy(inv_hbm.at[pl.ds(base, _CT)], i0_v)
    pltpu.sync_copy(inv_hbm.at[pl.ds(_NTOK + base, _CT)], i1_v)
    pltpu.sync_copy(wcat_hbm.at[pl.ds(base, _CT)], w0_v)
    pltpu.sync_copy(wcat_hbm.at[pl.ds(_NTOK + base, _CT)], w1_v)

    def _loop(c, _):
        off = base + c * _CC
        idx0 = i0_v[pl.ds(c * _CC, _CC)]
        idx1 = i1_v[pl.ds(c * _CC, _CC)]
        cp0 = pltpu.make_async_copy(ob_hbm.at[idx0], r0_v, sem)
        cp0.start()
        cp1 = pltpu.make_async_copy(ob_hbm.at[idx1], r1_v, sem)
        cp1.start()
        pltpu.sync_copy(x_hbm.at[pl.ds(off, _CC)], x_v)
        cp0.wait()
        cp1.wait()
        w0c = w0_v[pl.ds(c * _CC, _CC)]
        w1c = w1_v[pl.ds(c * _CC, _CC)]

        def _row(r, _):
            rl = jnp.full((16,), r, jnp.int32)
            w0bc = jnp.take_along_axis(w0c, rl, axis=0)
            w1bc = jnp.take_along_axis(w1c, rl, axis=0)

            def _col(j, _):
                for u in range(4):
                    sl = pl.ds(j * 64 + u * 16, 16)
                    y_v[r, sl] = (x_v[r, sl] + w0bc * r0_v[r, sl]
                                  + w1bc * r1_v[r, sl])
                return 0
            lax.fori_loop(0, _D // 64, _col, 0)
            return 0

        lax.fori_loop(0, _CC, _row, 0)
        pltpu.sync_copy(y_v, y_hbm.at[pl.ds(off, _CC)])
        return 0

    lax.fori_loop(0, _CT // _CC, _loop, 0)


def _combine(x_flat, ob, inv, wcat):
    mesh = plsc.VectorSubcoreMesh(
        core_axis_name="c", subcore_axis_name="s", num_cores=_NC,
        num_subcores=_NS)
    fn = pl.kernel(
        _combine_body,
        out_type=jax.ShapeDtypeStruct((_NTOK, _D), jnp.float32),
        mesh=mesh,
        scratch_types=[
            pltpu.VMEM((_CT,), jnp.int32),
            pltpu.VMEM((_CT,), jnp.int32),
            pltpu.VMEM((_CT,), jnp.float32),
            pltpu.VMEM((_CT,), jnp.float32),
            pltpu.VMEM((_CC, _D), jnp.float32),
            pltpu.VMEM((_CC, _D), jnp.float32),
            pltpu.VMEM((_CC, _D), jnp.float32),
            pltpu.VMEM((_CC, _D), jnp.float32),
            pltpu.SemaphoreType.DMA,
        ],
        compiler_params=pltpu.CompilerParams(needs_layout_passes=False),
    )
    return fn(x_flat, ob, inv, wcat)


def kernel(x, gamma, beta, Wg, W1, b1, W2, b2):
    Bx, Lx, Dx = x.shape
    x_flat = x.reshape(_NTOK, _D)
    xn, e0, e1, w0, w1, acc = _gate(x_flat, gamma, beta, Wg)

    ecat = jnp.concatenate([e0.reshape(-1), e1.reshape(-1)])
    wcat = jnp.concatenate([w0.reshape(-1), w1.reshape(-1)])
    hist = _hist(ecat)
    xs, inv = _dispatch(ecat, xn, hist)
    counts = jnp.sum(hist, axis=0)

    padded = ((counts + (_T - 1)) // _T) * _T
    ends = jnp.cumsum(padded)
    tvec = jnp.arange(_NT, dtype=jnp.int32) * _T
    te = jnp.sum((tvec[:, None] >= ends[None, :]).astype(jnp.int32), axis=1)
    te = jnp.minimum(te, _E - 1).astype(jnp.int32)

    ob = _ffn(te, xs, W1, b1.reshape(_E, 1, _FF),
              W2, b2.reshape(_E, 1, _D))

    y = _combine(x_flat, ob, inv, wcat).reshape(Bx, Lx, Dx)

    ps = acc[0, :_E]
    top1 = acc[1, :_E]
    zsum = acc[2, 0]
    aux = (_E * jnp.sum((top1 / _NTOK) * (ps / _NTOK)) * _LB_COEF
           + (zsum / _NTOK) * _Z_COEF)
    return y, aux


# speedup vs baseline: 1.3618x; 1.3618x over previous
import functools

import jax
import jax.numpy as jnp
from jax import lax
from jax.experimental import pallas as pl
from jax.experimental.pallas import tpu as pltpu
from jax.experimental.pallas import tpu_sc as plsc

_D = 1024
_E = 16
_FF = 2048
_K = 2
_NTOK = 4096
_NSLOT = _K * _NTOK
_T = 128
_NPAD = _NSLOT + _E * _T
_NT = _NPAD // _T
_TT = 256
_NTT = _NTOK // _TT
_LB_COEF = 0.01
_Z_COEF = 0.001

_NC = 2
_NS = 16


def _gate_body(x_ref, g_ref, b_ref, wg_ref,
               xn_ref, e0_ref, e1_ref, w0_ref, w1_ref, acc_ref):
    t = pl.program_id(0)
    xt = x_ref[...]
    mu = jnp.mean(xt, axis=1, keepdims=True)
    var = jnp.mean((xt - mu) ** 2, axis=1, keepdims=True)
    xn = g_ref[...] * (xt - mu) / jnp.sqrt(var + 1e-5) + b_ref[...]
    xn_ref[...] = xn
    logits = lax.dot_general(xn, wg_ref[...], (((1,), (1,)), ((), ())),
                             preferred_element_type=jnp.float32)
    logits = jnp.clip(logits, -10.0, 10.0)
    lm = jnp.max(logits, axis=1, keepdims=True)
    ex = jnp.exp(logits - lm)
    se = jnp.sum(ex, axis=1, keepdims=True)
    probs = ex / se
    lse = jnp.log(se[:, 0]) + lm[:, 0]
    iota = lax.broadcasted_iota(jnp.int32, (_TT, _E), 1)
    m0 = jnp.max(probs, axis=1, keepdims=True)
    e0 = jnp.min(jnp.where(probs == m0, iota, _E), axis=1)
    pm = jnp.where(iota == e0[:, None], -1.0, probs)
    m1 = jnp.max(pm, axis=1, keepdims=True)
    e1 = jnp.min(jnp.where(pm == m1, iota, _E), axis=1)
    denom = m0[:, 0] + m1[:, 0] + 1e-8
    e0_ref[0, 0, :] = e0
    e1_ref[0, 0, :] = e1
    w0_ref[0, 0, :] = m0[:, 0] / denom
    w1_ref[0, 0, :] = m1[:, 0] / denom
    ps = jnp.sum(probs, axis=0)
    top1 = jnp.sum((iota == e0[:, None]).astype(jnp.float32), axis=0)
    zsq = jnp.sum(lse * lse)
    zpad = jnp.zeros((128 - _E,), jnp.float32)
    col = lax.broadcasted_iota(jnp.int32, (1, 128), 1)
    part = jnp.concatenate([
        jnp.concatenate([ps, zpad]).reshape(1, 128),
        jnp.concatenate([top1, zpad]).reshape(1, 128),
        jnp.where(col == 0, zsq, 0.0),
        jnp.zeros((5, 128), jnp.float32),
    ], axis=0)

    @pl.when(t == 0)
    def _():
        acc_ref[...] = part

    @pl.when(t != 0)
    def _():
        acc_ref[...] = acc_ref[...] + part


def _gate(x_flat, gamma, beta, wg):
    return pl.pallas_call(
        _gate_body,
        grid=(_NTT,),
        in_specs=[
            pl.BlockSpec((_TT, _D), lambda t: (t, 0)),
            pl.BlockSpec((1, _D), lambda t: (0, 0)),
            pl.BlockSpec((1, _D), lambda t: (0, 0)),
            pl.BlockSpec((_E, _D), lambda t: (0, 0)),
        ],
        out_specs=[
            pl.BlockSpec((_TT, _D), lambda t: (t, 0)),
            pl.BlockSpec((1, 1, _TT), lambda t: (t, 0, 0)),
            pl.BlockSpec((1, 1, _TT), lambda t: (t, 0, 0)),
            pl.BlockSpec((1, 1, _TT), lambda t: (t, 0, 0)),
            pl.BlockSpec((1, 1, _TT), lambda t: (t, 0, 0)),
            pl.BlockSpec((8, 128), lambda t: (0, 0)),
        ],
        out_shape=[
            jax.ShapeDtypeStruct((_NTOK, _D), jnp.float32),
            jax.ShapeDtypeStruct((_NTT, 1, _TT), jnp.int32),
            jax.ShapeDtypeStruct((_NTT, 1, _TT), jnp.int32),
            jax.ShapeDtypeStruct((_NTT, 1, _TT), jnp.float32),
            jax.ShapeDtypeStruct((_NTT, 1, _TT), jnp.float32),
            jax.ShapeDtypeStruct((8, 128), jnp.float32),
        ],
    )(x_flat, gamma.reshape(1, _D), beta.reshape(1, _D), wg)


_NW = _NC * _NS
_RS = _NSLOT // _NW
_DC = 16
_NDC = _RS // _DC
_NBUF = 4


def _hist_body(ecat, hist_hbm, e_v, hist_v):
    wid = lax.axis_index("s") * _NC + lax.axis_index("c")
    pltpu.sync_copy(ecat.at[pl.ds(wid * _RS, _RS)], e_v)
    iota16 = lax.iota(jnp.int32, 16)
    lane15 = jnp.full((16,), 15, jnp.int32)

    def _splat_last(v):
        return jnp.take_along_axis(v, lane15, axis=0)

    def _h(j, hist):
        ev = e_v[pl.ds(j * 16, 16)]
        for b in range(_E):
            cs = plsc.cumsum(jnp.where(ev == b, 1, 0))
            hist = hist + jnp.where(iota16 == b, _splat_last(cs), 0)
        return hist

    hist = lax.fori_loop(0, _RS // 16, _h, jnp.zeros((16,), jnp.int32))
    hist_v[...] = hist
    pltpu.sync_copy(hist_v, hist_hbm.at[wid])


def _hist(ecat):
    mesh = plsc.VectorSubcoreMesh(
        core_axis_name="c", subcore_axis_name="s", num_cores=_NC,
        num_subcores=_NS)
    fn = pl.kernel(
        _hist_body,
        out_type=jax.ShapeDtypeStruct((_NW, 16), jnp.int32),
        mesh=mesh,
        scratch_types=[
            pltpu.VMEM((_RS,), jnp.int32),
            pltpu.VMEM((16,), jnp.int32),
        ],
        compiler_params=pltpu.CompilerParams(needs_layout_passes=False),
    )
    return fn(ecat)


def _dispatch_body(ecat, xn_hbm, hist_hbm, xs_hbm, inv_hbm,
                   e_v, pos_v, allh_v, posh_v, rows0, rows1, rows2, rows3,
                   l0, l1, l2, l3, s0, s1, s2, s3):
    wid = lax.axis_index("s") * _NC + lax.axis_index("c")
    base = wid * _RS
    pltpu.sync_copy(ecat.at[pl.ds(base, _RS)], e_v)
    pltpu.sync_copy(hist_hbm, allh_v)

    iota16 = lax.iota(jnp.int32, 16)
    lane15 = jnp.full((16,), 15, jnp.int32)

    def _splat_last(v):
        return jnp.take_along_axis(v, lane15, axis=0)

    def _s(r, carry):
        tot, pre = carry
        h = allh_v[r, :]
        m = jnp.where(r < wid, 1, 0)
        return tot + h, pre + h * m

    tot, pre = lax.fori_loop(
        0, _NW, _s,
        (jnp.zeros((16,), jnp.int32), jnp.zeros((16,), jnp.int32)))
    padded = ((tot + (_T - 1)) // _T) * _T
    offs_incl = plsc.cumsum(padded)
    start = offs_incl - padded + pre

    def _pz(j, _):
        pos_v[pl.ds(j * 16, 16)] = jnp.zeros((16,), jnp.int32)
        return 0
    lax.fori_loop(0, _RS // 16, _pz, 0)

    for b in range(_E):
        s_b = jnp.take_along_axis(start, jnp.full((16,), b, jnp.int32),
                                  axis=0)

        def _r(j, carry, b=b, s_b=s_b):
            sl = pl.ds(j * 16, 16)
            m = e_v[sl] == b
            cs = plsc.cumsum(jnp.where(m, 1, 0))
            pos_v[sl] = jnp.where(m, s_b + carry + cs - 1, pos_v[sl])
            return carry + _splat_last(cs)

        lax.fori_loop(0, _RS // 16, _r, jnp.zeros((16,), jnp.int32))

    for ch in range(_NDC):
        def _c(j, _, ch=ch):
            posh_v[ch, pl.ds(j * 16, 16)] = pos_v[pl.ds(ch * _DC + j * 16, 16)]
            return 0
        lax.fori_loop(0, _DC // 16, _c, 0)

    tokbase = base % _NTOK

    def _ld(ch, rows, sem):
        return pltpu.make_async_copy(
            xn_hbm.at[pl.ds(tokbase + ch * _DC, _DC)], rows, sem)

    def _sc(ch, rows, sem):
        return pltpu.make_async_copy(rows, xs_hbm.at[posh_v.at[ch]], sem)

    bufs = ((rows0, l0, s0), (rows1, l1, s1), (rows2, l2, s2),
            (rows3, l3, s3))
    for b, (rows, l, s) in enumerate(bufs):
        _ld(b, rows, l).start()

    def _grp(q, _):
        c_base = _NBUF * q
        for b, (rows, l, s) in enumerate(bufs):
            _ld(c_base + b, rows, l).wait()
            _sc(c_base + b, rows, s).start()

        @pl.when(q < _NDC // _NBUF - 1)
        def _():
            for b, (rows, l, s) in enumerate(bufs):
                _sc(c_base + b, rows, s).wait()
                _ld(c_base + b + _NBUF, rows, l).start()
        return 0

    lax.fori_loop(0, _NDC // _NBUF, _grp, 0)
    for b, (rows, l, s) in enumerate(bufs):
        _sc(_NDC - _NBUF + b, rows, s).wait()

    pltpu.sync_copy(pos_v, inv_hbm.at[pl.ds(base, _RS)])


def _dispatch(ecat, xn, hist):
    mesh = plsc.VectorSubcoreMesh(
        core_axis_name="c", subcore_axis_name="s", num_cores=_NC,
        num_subcores=_NS)
    fn = pl.kernel(
        _dispatch_body,
        out_type=(
            jax.ShapeDtypeStruct((_NPAD, _D), jnp.float32),
            jax.ShapeDtypeStruct((_NSLOT,), jnp.int32),
        ),
        mesh=mesh,
        scratch_types=[
            pltpu.VMEM((_RS,), jnp.int32),
            pltpu.VMEM((_RS,), jnp.int32),
            pltpu.VMEM((_NW, 16), jnp.int32),
            pltpu.VMEM((_NDC, _DC), jnp.int32),
            pltpu.VMEM((_DC, _D), jnp.float32),
            pltpu.VMEM((_DC, _D), jnp.float32),
            pltpu.VMEM((_DC, _D), jnp.float32),
            pltpu.VMEM((_DC, _D), jnp.float32),
            pltpu.SemaphoreType.DMA,
            pltpu.SemaphoreType.DMA,
            pltpu.SemaphoreType.DMA,
            pltpu.SemaphoreType.DMA,
            pltpu.SemaphoreType.DMA,
            pltpu.SemaphoreType.DMA,
            pltpu.SemaphoreType.DMA,
            pltpu.SemaphoreType.DMA,
        ],
        compiler_params=pltpu.CompilerParams(needs_layout_passes=False),
    )
    return fn(ecat, xn, hist)


def _ffn_body(te_ref, xs_ref, w1_ref, b1_ref, w2_ref, b2_ref, out_ref):
    del te_ref
    a = lax.dot_general(xs_ref[...], w1_ref[0], (((1,), (0,)), ((), ())),
                        preferred_element_type=jnp.float32)
    ap = a + b1_ref[0]
    h = 0.5 * ap * (1.0 + lax.erf(ap * 0.7071067811865476))
    o = lax.dot_general(h, w2_ref[0], (((1,), (0,)), ((), ())),
                        preferred_element_type=jnp.float32)
    out_ref[...] = o + b2_ref[0]


def _ffn(te, xs, w1, b1r, w2, b2r):
    grid_spec = pltpu.PrefetchScalarGridSpec(
        num_scalar_prefetch=1,
        grid=(_NT,),
        in_specs=[
            pl.BlockSpec((_T, _D), lambda t, te: (t, 0)),
            pl.BlockSpec((1, _D, _FF), lambda t, te: (te[t], 0, 0)),
            pl.BlockSpec((1, 1, _FF), lambda t, te: (te[t], 0, 0)),
            pl.BlockSpec((1, _FF, _D), lambda t, te: (te[t], 0, 0)),
            pl.BlockSpec((1, 1, _D), lambda t, te: (te[t], 0, 0)),
        ],
        out_specs=pl.BlockSpec((_T, _D), lambda t, te: (t, 0)),
    )
    return pl.pallas_call(
        _ffn_body,
        grid_spec=grid_spec,
        out_shape=jax.ShapeDtypeStruct((_NPAD, _D), jnp.float32),
    )(te, xs, w1, b1r, w2, b2r)


_CT = _NTOK // (_NC * _NS)
_CC = 16


def _combine_body(x_hbm, ob_hbm, inv_hbm, wcat_hbm, y_hbm,
                  i0_v, i1_v, w0_v, w1_v, x_v, r0_v, r1_v, y_v, sem):
    wid = lax.axis_index("s") * _NC + lax.axis_index("c")
    base = wid * _CT
    pltpu.sync_copy(inv_hbm.at[pl.ds(base, _CT)], i0_v)
    pltpu.sync_copy(inv_hbm.at[pl.ds(_NTOK + base, _CT)], i1_v)
    pltpu.sync_copy(wcat_hbm.at[pl.ds(base, _CT)], w0_v)
    pltpu.sync_copy(wcat_hbm.at[pl.ds(_NTOK + base, _CT)], w1_v)

    def _loop(c, _):
        off = base + c * _CC
        idx0 = i0_v[pl.ds(c * _CC, _CC)]
        idx1 = i1_v[pl.ds(c * _CC, _CC)]
        cp0 = pltpu.make_async_copy(ob_hbm.at[idx0], r0_v, sem)
        cp0.start()
        cp1 = pltpu.make_async_copy(ob_hbm.at[idx1], r1_v, sem)
        cp1.start()
        pltpu.sync_copy(x_hbm.at[pl.ds(off, _CC)], x_v)
        cp0.wait()
        cp1.wait()
        w0c = w0_v[pl.ds(c * _CC, _CC)]
        w1c = w1_v[pl.ds(c * _CC, _CC)]

        def _row(r, _):
            rl = jnp.full((16,), r, jnp.int32)
            w0bc = jnp.take_along_axis(w0c, rl, axis=0)
            w1bc = jnp.take_along_axis(w1c, rl, axis=0)

            def _col(j, _):
                for u in range(4):
                    sl = pl.ds(j * 64 + u * 16, 16)
                    y_v[r, sl] = (x_v[r, sl] + w0bc * r0_v[r, sl]
                                  + w1bc * r1_v[r, sl])
                return 0
            lax.fori_loop(0, _D // 64, _col, 0)
            return 0

        lax.fori_loop(0, _CC, _row, 0)
        pltpu.sync_copy(y_v, y_hbm.at[pl.ds(off, _CC)])
        return 0

    lax.fori_loop(0, _CT // _CC, _loop, 0)


def _combine(x_flat, ob, inv, wcat):
    mesh = plsc.VectorSubcoreMesh(
        core_axis_name="c", subcore_axis_name="s", num_cores=_NC,
        num_subcores=_NS)
    fn = pl.kernel(
        _combine_body,
        out_type=jax.ShapeDtypeStruct((_NTOK, _D), jnp.float32),
        mesh=mesh,
        scratch_types=[
            pltpu.VMEM((_CT,), jnp.int32),
            pltpu.VMEM((_CT,), jnp.int32),
            pltpu.VMEM((_CT,), jnp.float32),
            pltpu.VMEM((_CT,), jnp.float32),
            pltpu.VMEM((_CC, _D), jnp.float32),
            pltpu.VMEM((_CC, _D), jnp.float32),
            pltpu.VMEM((_CC, _D), jnp.float32),
            pltpu.VMEM((_CC, _D), jnp.float32),
            pltpu.SemaphoreType.DMA,
        ],
        compiler_params=pltpu.CompilerParams(needs_layout_passes=False),
    )
    return fn(x_flat, ob, inv, wcat)


def kernel(x, gamma, beta, Wg, W1, b1, W2, b2):
    Bx, Lx, Dx = x.shape
    x_flat = x.reshape(_NTOK, _D)
    xn, e0, e1, w0, w1, acc = _gate(x_flat, gamma, beta, Wg)

    ecat = jnp.concatenate([e0.reshape(-1), e1.reshape(-1)])
    wcat = jnp.concatenate([w0.reshape(-1), w1.reshape(-1)])
    hist = _hist(ecat)
    xs, inv = _dispatch(ecat, xn, hist)
    counts = jnp.sum(hist, axis=0)

    padded = ((counts + (_T - 1)) // _T) * _T
    ends = jnp.cumsum(padded)
    tvec = jnp.arange(_NT, dtype=jnp.int32) * _T
    te = jnp.sum((tvec[:, None] >= ends[None, :]).astype(jnp.int32), axis=1)
    te = jnp.minimum(te, _E - 1).astype(jnp.int32)

    ob = _ffn(te, xs, W1, b1.reshape(_E, 1, _FF),
              W2, b2.reshape(_E, 1, _D))

    y = _combine(x_flat, ob, inv, wcat).reshape(Bx, Lx, Dx)

    ps = acc[0, :_E]
    top1 = acc[1, :_E]
    zsum = acc[2, 0]
    aux = (_E * jnp.sum((top1 / _NTOK) * (ps / _NTOK)) * _LB_COEF
           + (zsum / _NTOK) * _Z_COEF)
    return y, aux
